# baseline (speedup 1.0000x reference)
import functools

import jax
import jax.numpy as jnp
from jax import lax
from jax.experimental import pallas as pl
from jax.experimental.pallas import tpu as pltpu
from jax.experimental.pallas import tpu_sc as plsc

_D = 128
_NC = 2
_NS = 16
_NW = _NC * _NS
_K = 128
_CHUNK = 8
_RPT = 632
_NACC = _NS * _RPT
_DUMMY = 10000


def _proj_body(x_ref, wt_ref, b_ref, o_ref):
    o_ref[...] = (
        jnp.dot(x_ref[...], wt_ref[...], preferred_element_type=jnp.float32)
        + b_ref[...]
    )


def _project(H, Wt, b2):
    M = H.shape[0]
    B = 2000
    return pl.pallas_call(
        _proj_body,
        grid=(M // B,),
        in_specs=[
            pl.BlockSpec((B, _D), lambda i: (i, 0)),
            pl.BlockSpec((_D, _D), lambda i: (0, 0)),
            pl.BlockSpec((1, _D), lambda i: (0, 0)),
        ],
        out_specs=pl.BlockSpec((B, _D), lambda i: (i, 0)),
        out_shape=jax.ShapeDtypeStruct((M, _D), jnp.float32),
    )(H, Wt, b2)


def _make_sc_agg(NB):
    mesh = plsc.VectorSubcoreMesh(
        core_axis_name="c", subcore_axis_name="s",
        num_cores=_NC, num_subcores=_NS,
    )
    sum_t = jax.ShapeDtypeStruct((_NC * _NACC, _D), jnp.float32)
    deg_t = jax.ShapeDtypeStruct((_NW, _NACC), jnp.float32)

    @functools.partial(
        pl.kernel,
        out_type=[sum_t, sum_t, sum_t, deg_t, deg_t, deg_t],
        mesh=mesh,
        compiler_params=pltpu.CompilerParams(needs_layout_passes=False),
        scratch_types=[
            pltpu.VMEM_SHARED((_NACC, _D), jnp.float32),
            pltpu.VMEM((_NACC,), jnp.float32),
            pltpu.VMEM((_CHUNK, _K), jnp.int32),
            pltpu.VMEM((_CHUNK, _K), jnp.int32),
            pltpu.VMEM((_CHUNK, _K), jnp.int32),
            pltpu.VMEM((_CHUNK, _K), jnp.int32),
            pltpu.VMEM((_K, _D), jnp.float32),
            pltpu.VMEM((_K, _D), jnp.float32),
            pltpu.SemaphoreType.DMA,
            pltpu.SemaphoreType.DMA,
        ],
    )
    def agg(t_op, t_m, s_seq, d_seq, s_m, d_m,
            st1, dt1, st2, dt2, st3, dt3, zrow, zdeg,
            o1, o2, o3, g1, g2, g3,
            acc, ldeg, iv_s0, iv_d0, iv_s1, iv_d1, rows_a, rows_b,
            sem_a, sem_b):
        cid = lax.axis_index("c")
        sid = lax.axis_index("s")
        wid = cid * _NS + sid
        base = sid * _RPT
        obase = cid * _NACC + base
        ones16 = jnp.full((16,), 1.0, jnp.float32)
        npair = _CHUNK // 2

        def one_pass(table, src_h, dst_h, stail_h, dtail_h, out_h, deg_h):
            pltpu.sync_copy(zrow, acc.at[pl.ds(base, _RPT)])
            pltpu.sync_copy(zdeg, ldeg)
            plsc.subcore_barrier()

            def deg_adds(ivd, b):
                for j in range(_K // 16):
                    i16 = ivd[b, pl.ds(j * 16, 16)]
                    plsc.addupdate_scatter(ldeg, [i16], ones16)

            def do_block(ivs, ivd, b, rows, sem):
                pltpu.make_async_copy(table.at[ivs.at[b]], rows, sem).wait()
                pltpu.sync_copy(rows, acc.at[ivd.at[b]], add=True)

            def run_chunk(ivs, ivd, tail_fire):
                def pair(b2, cc):
                    b = 2 * b2
                    pltpu.async_copy(table.at[ivs.at[b + 1]], rows_b, sem_b)
                    deg_adds(ivd, b)
                    do_block(ivs, ivd, b, rows_a, sem_a)
                    pltpu.async_copy(table.at[ivs.at[b + 2]], rows_a, sem_a)
                    deg_adds(ivd, b + 1)
                    do_block(ivs, ivd, b + 1, rows_b, sem_b)
                    return cc

                lax.fori_loop(0, npair - 1, pair, 0)
                b = _CHUNK - 2
                pltpu.async_copy(table.at[ivs.at[b + 1]], rows_b, sem_b)
                deg_adds(ivd, b)
                do_block(ivs, ivd, b, rows_a, sem_a)
                tail_fire()
                deg_adds(ivd, b + 1)
                do_block(ivs, ivd, b + 1, rows_b, sem_b)

            def stage(c, ivs, ivd):
                ibase = wid * NB + c * _CHUNK

                @pl.when(wid < _NW - 1)
                def _():
                    pltpu.sync_copy(src_h.at[pl.ds(ibase, _CHUNK)], ivs)
                    pltpu.sync_copy(dst_h.at[pl.ds(ibase, _CHUNK)], ivd)

                @pl.when(wid == _NW - 1)
                def _():
                    pltpu.sync_copy(stail_h.at[pl.ds(c * _CHUNK, _CHUNK)],
                                    ivs)
                    pltpu.sync_copy(dtail_h.at[pl.ds(c * _CHUNK, _CHUNK)],
                                    ivd)

            stage(0, iv_s0, iv_d0)
            pltpu.async_copy(table.at[iv_s0.at[0]], rows_a, sem_a)

            def outer(t, carry):
                c0 = 2 * t
                stage(c0 + 1, iv_s1, iv_d1)
                run_chunk(iv_s0, iv_d0, lambda: pltpu.async_copy(
                    table.at[iv_s1.at[0]], rows_a, sem_a))

                @pl.when(t < NB // (2 * _CHUNK) - 1)
                def _():
                    stage(c0 + 2, iv_s0, iv_d0)

                last = t == NB // (2 * _CHUNK) - 1

                def tail1():
                    @pl.when(jnp.logical_not(last))
                    def _():
                        pltpu.async_copy(table.at[iv_s0.at[0]],
                                         rows_a, sem_a)

                run_chunk(iv_s1, iv_d1, tail1)
                return carry

            lax.fori_loop(0, NB // (2 * _CHUNK), outer, 0)
            plsc.subcore_barrier()
            pltpu.sync_copy(acc.at[pl.ds(base, _RPT)],
                            out_h.at[pl.ds(obase, _RPT)])
            pltpu.sync_copy(ldeg, deg_h.at[wid])
            plsc.subcore_barrier()

        one_pass(t_op, s_seq, d_seq, st1, dt1, o1, g1)
        one_pass(t_op, s_m, d_m, st2, dt2, o2, g2)
        one_pass(t_m, d_m, s_m, st3, dt3, o3, g3)

    return agg


def _agg_term(s_ref, e_ref):
    deg = jnp.maximum(jnp.sum(e_ref[...], axis=0), 1.0)[:, None]
    return (s_ref[0] + s_ref[1]) / deg


def _comb_body(po_ref, s1_ref, e1_ref, s3_ref, e3_ref,
               pm_ref, s2_ref, e2_ref, oo_ref, om_ref):
    oo_ref[...] = jnp.maximum(
        po_ref[...] + _agg_term(s1_ref, e1_ref) + _agg_term(s3_ref, e3_ref),
        0.0)
    om_ref[...] = jnp.maximum(pm_ref[...] + _agg_term(s2_ref, e2_ref), 0.0)


def _combine(P_op, o1, g1, o3, g3, P_m, o2, g2):
    M = P_op.shape[0]
    B = 2048
    spec_p = pl.BlockSpec((B, _D), lambda i: (i, 0))
    spec_s = pl.BlockSpec((_NC, B, _D), lambda i: (0, i, 0))
    spec_e = pl.BlockSpec((_NW, B), lambda i: (0, i))
    return pl.pallas_call(
        _comb_body,
        grid=(pl.cdiv(M, B),),
        in_specs=[spec_p, spec_s, spec_e, spec_s, spec_e,
                  spec_p, spec_s, spec_e],
        out_specs=[spec_p, spec_p],
        out_shape=[jax.ShapeDtypeStruct((M, _D), jnp.float32)] * 2,
    )(P_op, o1, g1, o3, g3, P_m, o2, g2)


def kernel(H_op, H_m, E_seq, E_op2m, W_op, b_op, W_m, b_m):
    E = E_seq.shape[1]
    NB = -(-E // (_NW * _K))
    NB = -(-NB // (2 * _CHUNK)) * (2 * _CHUNK)
    E_pad = _NW * NB * _K

    assert E % _K == 0, "edge count must be a multiple of the block size"
    npad = E_pad - E
    pad_src = ((jnp.arange(npad, dtype=jnp.int32) * 79) %
               jnp.int32(10000)).reshape(-1, _K)
    pad_dst = (_DUMMY + jnp.arange(npad, dtype=jnp.int32) %
               (_NACC - _DUMMY)).reshape(-1, _K)
    last_base = (_NW - 1) * NB

    def raw(idx):
        return idx.reshape(E // _K, _K)

    def tail(idx, pad):
        return jnp.concatenate([raw(idx)[last_base:], pad])

    s_seq, d_seq = raw(E_seq[0]), raw(E_seq[1])
    s_m, d_m = raw(E_op2m[0]), raw(E_op2m[1])
    st1, dt1 = tail(E_seq[0], pad_src), tail(E_seq[1], pad_dst)
    st2, dt2 = tail(E_op2m[0], pad_src), tail(E_op2m[1], pad_dst)
    st3, dt3 = tail(E_op2m[1], pad_src), tail(E_op2m[0], pad_dst)

    zrow = jnp.zeros((_RPT, _D), jnp.float32)
    zdeg = jnp.zeros((_NACC,), jnp.float32)

    P_op = _project(H_op, W_op.T, b_op.reshape(1, _D))
    P_m = _project(H_m, W_m.T, b_m.reshape(1, _D))

    o1, o2, o3, g1, g2, g3 = _make_sc_agg(NB)(
        P_op, P_m, s_seq, d_seq, s_m, d_m,
        st1, dt1, st2, dt2, st3, dt3, zrow, zdeg)

    o1 = o1.reshape(_NC, _NACC, _D)
    o2 = o2.reshape(_NC, _NACC, _D)
    o3 = o3.reshape(_NC, _NACC, _D)

    H_op_new, H_m_new = _combine(P_op, o1, g1, o3, g3, P_m, o2, g2)
    return (H_op_new, H_m_new)


# speedup vs baseline: 1.0124x; 1.0124x over previous
import functools

import jax
import jax.numpy as jnp
from jax import lax
from jax.experimental import pallas as pl
from jax.experimental.pallas import tpu as pltpu
from jax.experimental.pallas import tpu_sc as plsc

_D = 128
_NC = 2
_NS = 16
_NW = _NC * _NS
_K = 128
_CHUNK = 8
_RPT = 632
_NACC = _NS * _RPT
_DUMMY = 10000


def _proj_body(x1_ref, w1_ref, b1_ref, x2_ref, w2_ref, b2_ref,
               o1_ref, o2_ref):
    o1_ref[...] = (
        jnp.dot(x1_ref[...], w1_ref[...], preferred_element_type=jnp.float32)
        + b1_ref[...]
    )
    o2_ref[...] = (
        jnp.dot(x2_ref[...], w2_ref[...], preferred_element_type=jnp.float32)
        + b2_ref[...]
    )


def _project(H1, W1t, b1, H2, W2t, b2):
    M = H1.shape[0]
    B = 2000
    spec_x = pl.BlockSpec((B, _D), lambda i: (i, 0))
    spec_w = pl.BlockSpec((_D, _D), lambda i: (0, 0))
    spec_b = pl.BlockSpec((1, _D), lambda i: (0, 0))
    return pl.pallas_call(
        _proj_body,
        grid=(M // B,),
        in_specs=[spec_x, spec_w, spec_b, spec_x, spec_w, spec_b],
        out_specs=[spec_x, spec_x],
        out_shape=[jax.ShapeDtypeStruct((M, _D), jnp.float32)] * 2,
    )(H1, W1t, b1, H2, W2t, b2)


def _make_sc_agg(NB):
    mesh = plsc.VectorSubcoreMesh(
        core_axis_name="c", subcore_axis_name="s",
        num_cores=_NC, num_subcores=_NS,
    )
    sum_t = jax.ShapeDtypeStruct((_NC * _NACC, _D), jnp.float32)
    deg_t = jax.ShapeDtypeStruct((_NW, _NACC), jnp.float32)

    @functools.partial(
        pl.kernel,
        out_type=[sum_t, sum_t, sum_t, deg_t, deg_t, deg_t],
        mesh=mesh,
        compiler_params=pltpu.CompilerParams(needs_layout_passes=False),
        scratch_types=[
            pltpu.VMEM_SHARED((_NACC, _D), jnp.float32),
            pltpu.VMEM((_NACC,), jnp.float32),
            pltpu.VMEM((_CHUNK, _K), jnp.int32),
            pltpu.VMEM((_CHUNK, _K), jnp.int32),
            pltpu.VMEM((_CHUNK, _K), jnp.int32),
            pltpu.VMEM((_CHUNK, _K), jnp.int32),
            pltpu.VMEM((_K, _D), jnp.float32),
            pltpu.VMEM((_K, _D), jnp.float32),
            pltpu.SemaphoreType.DMA,
            pltpu.SemaphoreType.DMA,
        ],
    )
    def agg(t_op, t_m, s_seq, d_seq, s_m, d_m,
            st1, dt1, st2, dt2, st3, dt3, zrow, zdeg,
            o1, o2, o3, g1, g2, g3,
            acc, ldeg, iv_s0, iv_d0, iv_s1, iv_d1, rows_a, rows_b,
            sem_a, sem_b):
        cid = lax.axis_index("c")
        sid = lax.axis_index("s")
        wid = cid * _NS + sid
        base = sid * _RPT
        obase = cid * _NACC + base
        ones16 = jnp.full((16,), 1.0, jnp.float32)
        npair = _CHUNK // 2

        def one_pass(table, src_h, dst_h, stail_h, dtail_h, out_h, deg_h):
            pltpu.sync_copy(zrow, acc.at[pl.ds(base, _RPT)])
            pltpu.sync_copy(zdeg, ldeg)
            plsc.subcore_barrier()

            def deg_adds(ivd, b):
                for j in range(_K // 16):
                    i16 = ivd[b, pl.ds(j * 16, 16)]
                    plsc.addupdate_scatter(ldeg, [i16], ones16)

            def do_block(ivs, ivd, b, rows, sem):
                pltpu.make_async_copy(table.at[ivs.at[b]], rows, sem).wait()
                pltpu.sync_copy(rows, acc.at[ivd.at[b]], add=True)

            def run_chunk(ivs, ivd, tail_fire):
                def pair(b2, cc):
                    b = 2 * b2
                    pltpu.async_copy(table.at[ivs.at[b + 1]], rows_b, sem_b)
                    deg_adds(ivd, b)
                    do_block(ivs, ivd, b, rows_a, sem_a)
                    pltpu.async_copy(table.at[ivs.at[b + 2]], rows_a, sem_a)
                    deg_adds(ivd, b + 1)
                    do_block(ivs, ivd, b + 1, rows_b, sem_b)
                    return cc

                lax.fori_loop(0, npair - 1, pair, 0)
                b = _CHUNK - 2
                pltpu.async_copy(table.at[ivs.at[b + 1]], rows_b, sem_b)
                deg_adds(ivd, b)
                do_block(ivs, ivd, b, rows_a, sem_a)
                tail_fire()
                deg_adds(ivd, b + 1)
                do_block(ivs, ivd, b + 1, rows_b, sem_b)

            def stage(c, ivs, ivd):
                ibase = wid * NB + c * _CHUNK

                @pl.when(wid < _NW - 1)
                def _():
                    pltpu.sync_copy(src_h.at[pl.ds(ibase, _CHUNK)], ivs)
                    pltpu.sync_copy(dst_h.at[pl.ds(ibase, _CHUNK)], ivd)

                @pl.when(wid == _NW - 1)
                def _():
                    pltpu.sync_copy(stail_h.at[pl.ds(c * _CHUNK, _CHUNK)],
                                    ivs)
                    pltpu.sync_copy(dtail_h.at[pl.ds(c * _CHUNK, _CHUNK)],
                                    ivd)

            stage(0, iv_s0, iv_d0)
            pltpu.async_copy(table.at[iv_s0.at[0]], rows_a, sem_a)

            def outer(t, carry):
                c0 = 2 * t
                stage(c0 + 1, iv_s1, iv_d1)
                run_chunk(iv_s0, iv_d0, lambda: pltpu.async_copy(
                    table.at[iv_s1.at[0]], rows_a, sem_a))

                @pl.when(t < NB // (2 * _CHUNK) - 1)
                def _():
                    stage(c0 + 2, iv_s0, iv_d0)

                last = t == NB // (2 * _CHUNK) - 1

                def tail1():
                    @pl.when(jnp.logical_not(last))
                    def _():
                        pltpu.async_copy(table.at[iv_s0.at[0]],
                                         rows_a, sem_a)

                run_chunk(iv_s1, iv_d1, tail1)
                return carry

            lax.fori_loop(0, NB // (2 * _CHUNK), outer, 0)
            plsc.subcore_barrier()
            pltpu.sync_copy(acc.at[pl.ds(base, _RPT)],
                            out_h.at[pl.ds(obase, _RPT)])
            pltpu.sync_copy(ldeg, deg_h.at[wid])
            plsc.subcore_barrier()

        one_pass(t_op, s_seq, d_seq, st1, dt1, o1, g1)
        one_pass(t_op, s_m, d_m, st2, dt2, o2, g2)
        one_pass(t_m, d_m, s_m, st3, dt3, o3, g3)

    return agg


def _agg_term(s_ref, e_ref):
    deg = jnp.maximum(jnp.sum(e_ref[...], axis=0), 1.0)[:, None]
    return (s_ref[0] + s_ref[1]) / deg


def _comb_body(po_ref, s1_ref, e1_ref, s3_ref, e3_ref,
               pm_ref, s2_ref, e2_ref, oo_ref, om_ref):
    oo_ref[...] = jnp.maximum(
        po_ref[...] + _agg_term(s1_ref, e1_ref) + _agg_term(s3_ref, e3_ref),
        0.0)
    om_ref[...] = jnp.maximum(pm_ref[...] + _agg_term(s2_ref, e2_ref), 0.0)


def _combine(P_op, o1, g1, o3, g3, P_m, o2, g2):
    M = P_op.shape[0]
    B = 2048
    spec_p = pl.BlockSpec((B, _D), lambda i: (i, 0))
    spec_s = pl.BlockSpec((_NC, B, _D), lambda i: (0, i, 0))
    spec_e = pl.BlockSpec((_NW, B), lambda i: (0, i))
    return pl.pallas_call(
        _comb_body,
        grid=(pl.cdiv(M, B),),
        in_specs=[spec_p, spec_s, spec_e, spec_s, spec_e,
                  spec_p, spec_s, spec_e],
        out_specs=[spec_p, spec_p],
        out_shape=[jax.ShapeDtypeStruct((M, _D), jnp.float32)] * 2,
    )(P_op, o1, g1, o3, g3, P_m, o2, g2)


def kernel(H_op, H_m, E_seq, E_op2m, W_op, b_op, W_m, b_m):
    E = E_seq.shape[1]
    NB = -(-E // (_NW * _K))
    NB = -(-NB // (2 * _CHUNK)) * (2 * _CHUNK)
    E_pad = _NW * NB * _K

    assert E % _K == 0, "edge count must be a multiple of the block size"
    npad = E_pad - E
    pad_src = ((jnp.arange(npad, dtype=jnp.int32) * 79) %
               jnp.int32(10000)).reshape(-1, _K)
    pad_dst = (_DUMMY + jnp.arange(npad, dtype=jnp.int32) %
               (_NACC - _DUMMY)).reshape(-1, _K)
    last_base = (_NW - 1) * NB

    def raw(idx):
        return idx.reshape(E // _K, _K)

    def tail(idx, pad):
        return jnp.concatenate([raw(idx)[last_base:], pad])

    s_seq, d_seq = raw(E_seq[0]), raw(E_seq[1])
    s_m, d_m = raw(E_op2m[0]), raw(E_op2m[1])
    st1, dt1 = tail(E_seq[0], pad_src), tail(E_seq[1], pad_dst)
    st2, dt2 = tail(E_op2m[0], pad_src), tail(E_op2m[1], pad_dst)
    st3, dt3 = tail(E_op2m[1], pad_src), tail(E_op2m[0], pad_dst)

    zrow = jnp.zeros((_RPT, _D), jnp.float32)
    zdeg = jnp.zeros((_NACC,), jnp.float32)

    P_op, P_m = _project(H_op, W_op.T, b_op.reshape(1, _D),
                         H_m, W_m.T, b_m.reshape(1, _D))

    o1, o2, o3, g1, g2, g3 = _make_sc_agg(NB)(
        P_op, P_m, s_seq, d_seq, s_m, d_m,
        st1, dt1, st2, dt2, st3, dt3, zrow, zdeg)

    o1 = o1.reshape(_NC, _NACC, _D)
    o2 = o2.reshape(_NC, _NACC, _D)
    o3 = o3.reshape(_NC, _NACC, _D)

    H_op_new, H_m_new = _combine(P_op, o1, g1, o3, g3, P_m, o2, g2)
    return (H_op_new, H_m_new)
